# baseline (speedup 1.0000x reference)
import jax
import jax.numpy as jnp
from jax.experimental import pallas as pl
from jax.experimental.pallas import tpu as pltpu

_N_CHUNKS = 16


def _copy_kernel(src_ref, dst_ref, sems):
    rows = src_ref.shape[0]
    chunk = rows // _N_CHUNKS
    copies = [
        pltpu.make_async_copy(
            src_ref.at[pl.ds(i * chunk, chunk)],
            dst_ref.at[pl.ds(i * chunk, chunk)],
            sems.at[i],
        )
        for i in range(_N_CHUNKS)
    ]
    for c in copies:
        c.start()
    for c in copies:
        c.wait()


def kernel(pos_emb):
    return pl.pallas_call(
        _copy_kernel,
        out_shape=jax.ShapeDtypeStruct(pos_emb.shape, pos_emb.dtype),
        in_specs=[pl.BlockSpec(memory_space=pl.ANY)],
        out_specs=pl.BlockSpec(memory_space=pl.ANY),
        scratch_shapes=[pltpu.SemaphoreType.DMA((_N_CHUNKS,))],
    )(pos_emb)


# speedup vs baseline: 43.1910x; 43.1910x over previous
import jax
import jax.numpy as jnp
from jax.experimental import pallas as pl
from jax.experimental.pallas import tpu as pltpu

_BLOCK_ROWS = 256


def _copy_kernel(src_ref, dst_ref):
    dst_ref[...] = src_ref[...]


def kernel(pos_emb):
    rows, width = pos_emb.shape
    grid = (rows // _BLOCK_ROWS,)
    return pl.pallas_call(
        _copy_kernel,
        out_shape=jax.ShapeDtypeStruct(pos_emb.shape, pos_emb.dtype),
        grid=grid,
        in_specs=[pl.BlockSpec((_BLOCK_ROWS, width), lambda i: (i, 0))],
        out_specs=pl.BlockSpec((_BLOCK_ROWS, width), lambda i: (i, 0)),
    )(pos_emb)


# speedup vs baseline: 48.8170x; 1.1303x over previous
import jax
import jax.numpy as jnp
from jax.experimental import pallas as pl
from jax.experimental.pallas import tpu as pltpu

_BLOCK_ROWS = 1024


def _copy_kernel(src_ref, dst_ref):
    dst_ref[...] = src_ref[...]


def kernel(pos_emb):
    rows, width = pos_emb.shape
    grid = (rows // _BLOCK_ROWS,)
    return pl.pallas_call(
        _copy_kernel,
        out_shape=jax.ShapeDtypeStruct(pos_emb.shape, pos_emb.dtype),
        grid=grid,
        in_specs=[pl.BlockSpec((_BLOCK_ROWS, width), lambda i: (i, 0))],
        out_specs=pl.BlockSpec((_BLOCK_ROWS, width), lambda i: (i, 0)),
    )(pos_emb)
